# baseline (speedup 1.0000x reference)
import jax
import jax.numpy as jnp
from jax.experimental import pallas as pl
from jax.experimental.pallas import tpu as pltpu

S = 8192
M = 1024
E = 16
TOP_K = 2
CAP = TOP_K * ((S + E - 1) // E)
BS = 2048
NB = S // BS
HB = 1024
EPS = float(jnp.finfo(jnp.float32).eps)


def _fused_kernel(x_hbm, wg_ref, out_ref, loss_ref, xcache, route, c1, c2, meacc,
                  dma_sems):
    i = pl.program_id(0)

    @pl.when(i == 0)
    def _init():
        c1[...] = jnp.zeros_like(c1)
        c2[...] = jnp.zeros_like(c2)
        meacc[...] = jnp.zeros_like(meacc)
        for b in range(NB):
            pltpu.make_async_copy(
                x_hbm.at[pl.ds(b * BS, BS), :],
                xcache.at[pl.ds(b * BS, BS), :],
                dma_sems.at[b],
            ).start()

    @pl.when(i < NB)
    def _phase_route():
        pltpu.make_async_copy(
            x_hbm.at[pl.ds(i * BS, BS), :],
            xcache.at[pl.ds(i * BS, BS), :],
            dma_sems.at[i],
        ).wait()
        x = xcache[pl.ds(i * BS, BS), :]
        logits = jax.lax.dot_general(
            wg_ref[...], x, (((1,), (1,)), ((), ())),
            preferred_element_type=jnp.float32,
        )
        m0 = jnp.max(logits, axis=0, keepdims=True)
        ex = jnp.exp(logits - m0)
        gates = ex * (1.0 / jnp.sum(ex, axis=0, keepdims=True))

        rowi = jax.lax.broadcasted_iota(jnp.int32, (E, BS), 0)
        g0 = jnp.max(gates, axis=0, keepdims=True)
        a0 = jnp.min(jnp.where(gates == g0, rowi, E), axis=0, keepdims=True)
        mask1 = rowi == a0
        gates2 = jnp.where(mask1, -jnp.inf, gates)
        g1 = jnp.max(gates2, axis=0, keepdims=True)
        a1 = jnp.min(jnp.where(gates2 == g1, rowi, E), axis=0, keepdims=True)
        mask2 = rowi == a1

        denom = jnp.maximum(g0 + g1, EPS)
        g0n = g0 / denom
        g1n = g1 / denom

        m1f = mask1.astype(jnp.float32)
        m2f = mask2.astype(jnp.float32)
        m12 = jnp.concatenate([m1f, m2f], axis=0)
        r2 = jax.lax.broadcasted_iota(jnp.int32, (HB, HB), 0)
        c2i = jax.lax.broadcasted_iota(jnp.int32, (HB, HB), 1)
        triu = (r2 <= c2i).astype(jnp.bfloat16)
        m12b = m12.astype(jnp.bfloat16)
        parts = []
        sub_carry = None
        for h in range(BS // HB):
            csh = jnp.dot(m12b[:, h * HB : (h + 1) * HB], triu,
                          preferred_element_type=jnp.float32)
            if sub_carry is not None:
                csh = csh + sub_carry
            sub_carry = csh[:, HB - 1 : HB]
            parts.append(csh)
        cs12 = jnp.concatenate(parts, axis=1)
        cs1 = cs12[:E, :] + c1[...]
        cs2 = cs12[E:, :] + c2[...]

        loc1 = jnp.sum(cs1 * m1f, axis=0, keepdims=True) - 1.0
        valid1 = (loc1 < CAP).astype(jnp.float32)
        base = g0n * valid1
        loc2p = jnp.sum(cs2 * m2f, axis=0, keepdims=True) - 1.0

        c1[...] = cs1[:, BS - 1 : BS]
        c2[...] = cs2[:, BS - 1 : BS]
        meacc[...] = meacc[...] + gates

        rowi8 = jax.lax.broadcasted_iota(jnp.int32, (8, BS), 0)
        route[:, pl.ds(i * BS, BS)] = (
            base * (rowi8 == 0)
            + g1n * (rowi8 == 1)
            + loc2p * (rowi8 == 2)
            + a1.astype(jnp.float32) * (rowi8 == 3)
        )

        @pl.when(i == NB - 1)
        def _fin():
            me = jnp.sum(meacc[...], axis=1, keepdims=True)
            loss_ref[...] = (jnp.sum(me * c1[...]) * (E / (S * S))).reshape(1, 1)

    @pl.when(i >= NB)
    def _phase_scale():
        b = i - NB
        r = route[:, pl.ds(b * BS, BS)]
        base = r[0:1, :]
        g1n = r[1:2, :]
        loc2p = r[2:3, :]
        idx1 = r[3:4, :].astype(jnp.int32)
        rowi = jax.lax.broadcasted_iota(jnp.int32, (E, BS), 0)
        mask2 = (rowi == idx1).astype(jnp.float32)
        cnt_sel = jnp.sum(mask2 * c1[...], axis=0, keepdims=True)
        valid2 = ((loc2p + cnt_sel) < CAP).astype(jnp.float32)
        scale_t = base + g1n * valid2
        scale = jnp.transpose(scale_t, (1, 0))
        out_ref[...] = scale * xcache[pl.ds(b * BS, BS), :]


@jax.jit
def kernel(input, wg):
    x = input

    out, loss = pl.pallas_call(
        _fused_kernel,
        grid=(2 * NB,),
        in_specs=[
            pl.BlockSpec(memory_space=pltpu.MemorySpace.HBM),
            pl.BlockSpec((E, M), lambda i: (0, 0)),
        ],
        out_specs=[
            pl.BlockSpec((BS, M), lambda i: (jnp.where(i < NB, 0, i - NB), 0)),
            pl.BlockSpec((1, 1), lambda i: (0, 0)),
        ],
        out_shape=[
            jax.ShapeDtypeStruct((S, M), jnp.float32),
            jax.ShapeDtypeStruct((1, 1), jnp.float32),
        ],
        scratch_shapes=[
            pltpu.VMEM((S, M), jnp.float32),
            pltpu.VMEM((8, S), jnp.float32),
            pltpu.VMEM((E, 1), jnp.float32),
            pltpu.VMEM((E, 1), jnp.float32),
            pltpu.VMEM((E, BS), jnp.float32),
            pltpu.SemaphoreType.DMA((NB,)),
        ],
    )(x, wg)

    return out, loss[0, 0]
